# baseline (speedup 1.0000x reference)
import functools

import jax
import jax.numpy as jnp
from jax import lax
from jax.experimental import pallas as pl
from jax.experimental.pallas import tpu as pltpu
from jax.experimental.pallas import tpu_sc as plsc

H = 1024
F = 512
E = 8
FS = 1024
T = 2048
BT = 128
NBLK = T * 2 // BT + E
PADT = NBLK * BT
NW = 32
CW = (2 * T) // NW


def _silu(v):
    return v * jax.nn.sigmoid(v)


def _routing_body(x_ref, gwt_ref, gw_ref, bias_row_ref, bias_col_ref,
                  wa_ref, wb_ref, pos_ref, blk_e_ref, blk_v_ref):
    x = x_ref[...]

    logits = jnp.dot(x, gwt_ref[...], preferred_element_type=jnp.float32)
    scores = jax.nn.sigmoid(logits)
    sfc = scores + bias_row_ref[...]
    colE = lax.broadcasted_iota(jnp.int32, (T, E), 1)
    m1 = jnp.max(sfc, axis=1, keepdims=True)
    i1 = jnp.min(jnp.where(sfc == m1, colE, E), axis=1, keepdims=True)
    oh1 = (colE == i1).astype(jnp.float32)
    sfc2 = jnp.where(oh1 > 0, -jnp.inf, sfc)
    m2 = jnp.max(sfc2, axis=1, keepdims=True)
    i2 = jnp.min(jnp.where(sfc2 == m2, colE, E), axis=1, keepdims=True)
    oh2 = (colE == i2).astype(jnp.float32)
    s1 = jnp.sum(oh1 * scores, axis=1, keepdims=True)
    s2 = jnp.sum(oh2 * scores, axis=1, keepdims=True)
    den = s1 + s2 + 1e-20
    wa_ref[...] = s1 / den
    wb_ref[...] = s2 / den

    logitsT = lax.dot_general(gw_ref[...], x, (((1,), (1,)), ((), ())),
                              preferred_element_type=jnp.float32)
    scoresT = jax.nn.sigmoid(logitsT)
    sfcT = scoresT + bias_col_ref[...]
    rowE = lax.broadcasted_iota(jnp.int32, (E, T), 0)
    n1 = jnp.max(sfcT, axis=0, keepdims=True)
    j1 = jnp.min(jnp.where(sfcT == n1, rowE, E), axis=0, keepdims=True)
    p1 = (rowE == j1).astype(jnp.float32)
    sfcT2 = jnp.where(p1 > 0, -jnp.inf, sfcT)
    n2 = jnp.max(sfcT2, axis=0, keepdims=True)
    j2 = jnp.min(jnp.where(sfcT2 == n2, rowE, E), axis=0, keepdims=True)
    p2 = (rowE == j2).astype(jnp.float32)

    cnt = jnp.sum(p1 + p2, axis=1, keepdims=True)
    cnt_pad = jnp.floor((cnt + (BT - 1)) * (1.0 / BT)) * BT
    er = lax.broadcasted_iota(jnp.int32, (E, E), 0)
    ec = lax.broadcasted_iota(jnp.int32, (E, E), 1)
    lower = (er > ec).astype(jnp.float32)
    off = jnp.dot(lower, cnt_pad, preferred_element_type=jnp.float32)

    ohk = jnp.concatenate([p1[:, None, :], p2[:, None, :]], axis=1)
    oh4 = ohk.reshape(E, 2, T // BT, BT)
    cr = lax.broadcasted_iota(jnp.int32, (BT, BT), 0)
    cc = lax.broadcasted_iota(jnp.int32, (BT, BT), 1)
    l128 = (cr < cc).astype(jnp.float32)
    rankin = lax.dot_general(oh4, l128, (((3,), (0,)), ((), ())),
                             preferred_element_type=jnp.float32)
    tot = jnp.sum(oh4, axis=3)
    gr = lax.broadcasted_iota(jnp.int32, (T // BT, T // BT), 0)
    gc = lax.broadcasted_iota(jnp.int32, (T // BT, T // BT), 1)
    l16 = (gr < gc).astype(jnp.float32)
    exc_c = lax.dot_general(tot, l16, (((2,), (0,)), ((), ())))
    tk0 = jnp.sum(tot[:, 0:1, :], axis=2, keepdims=True)
    kof = lax.broadcasted_iota(jnp.int32, (E, 2, T // BT), 1).astype(
        jnp.float32)
    exc = exc_c + kof * tk0
    rank4 = rankin + exc[..., None]
    posf = jnp.sum(oh4 * (rank4 + off.reshape(E, 1, 1, 1)), axis=0)
    pos_ref[...] = posf.reshape(2, T).astype(jnp.int32)

    bstart = lax.broadcasted_iota(jnp.int32, (E, NBLK), 1).astype(
        jnp.float32) * BT
    offb = off
    inb = jnp.logical_and(bstart >= offb, bstart < offb + cnt_pad)
    inbf = inb.astype(jnp.float32)
    ev = lax.broadcasted_iota(jnp.int32, (E, NBLK), 0).astype(jnp.float32)
    blk_e_f = jnp.sum(ev * inbf, axis=0, keepdims=True)
    valid = jnp.sum(inbf, axis=0, keepdims=True)
    emax = jnp.max(blk_e_f, axis=1, keepdims=True)
    blk_e_ref[...] = jnp.where(valid > 0, blk_e_f, emax).astype(jnp.int32)
    blk_v_ref[...] = valid.astype(jnp.int32)


def _routing(x, gate_w, gate_bias):
    return pl.pallas_call(
        _routing_body,
        out_shape=(
            jax.ShapeDtypeStruct((T, 1), jnp.float32),
            jax.ShapeDtypeStruct((T, 1), jnp.float32),
            jax.ShapeDtypeStruct((2, T), jnp.int32),
            jax.ShapeDtypeStruct((1, NBLK), jnp.int32),
            jax.ShapeDtypeStruct((1, NBLK), jnp.int32),
        ),
    )(x, gate_w.T, gate_w, gate_bias.reshape(1, E), gate_bias.reshape(E, 1))


def _dispatch_sc(x, pos3):
    mesh = plsc.VectorSubcoreMesh(core_axis_name="c", subcore_axis_name="s")

    @functools.partial(
        pl.kernel,
        mesh=mesh,
        out_type=jax.ShapeDtypeStruct((PADT, H), jnp.float32),
        scratch_types=[
            pltpu.VMEM((2, CW // 2), jnp.int32),
            pltpu.VMEM((CW // 2, H), jnp.float32),
            pltpu.SemaphoreType.DMA,
        ],
    )
    def k(x_hbm, pos_hbm, disp_hbm, idx_v, buf_v, sem):
        w = lax.axis_index("s") * 2 + lax.axis_index("c")
        pltpu.sync_copy(pos_hbm.at[w], idx_v)
        for j in range(2):
            ebase = w * CW + j * (CW // 2)
            tb = lax.rem(ebase, T)
            pltpu.sync_copy(x_hbm.at[pl.ds(tb, CW // 2)], buf_v)
            pltpu.async_copy(buf_v, disp_hbm.at[idx_v.at[j]], sem).wait()

    return k(x, pos3)


def _ggemm_body(be_ref, bv_ref, d_ref, w1_ref, w2_ref, w3_ref, y_ref):
    b = pl.program_id(0)

    @pl.when(bv_ref[b] == 1)
    def _():
        xb = d_ref[...]
        h1 = jnp.dot(xb, w1_ref[0], preferred_element_type=jnp.float32)
        h3 = jnp.dot(xb, w3_ref[0], preferred_element_type=jnp.float32)
        y_ref[...] = jnp.dot(_silu(h1) * h3, w2_ref[0],
                             preferred_element_type=jnp.float32)


def _ggemm(disp, blk_e, blk_v, w1, w2, w3):
    grid_spec = pltpu.PrefetchScalarGridSpec(
        num_scalar_prefetch=2,
        grid=(NBLK,),
        in_specs=[
            pl.BlockSpec((BT, H), lambda b, be, bv: (b, 0)),
            pl.BlockSpec((1, H, F), lambda b, be, bv: (be[b], 0, 0)),
            pl.BlockSpec((1, F, H), lambda b, be, bv: (be[b], 0, 0)),
            pl.BlockSpec((1, H, F), lambda b, be, bv: (be[b], 0, 0)),
        ],
        out_specs=pl.BlockSpec((BT, H), lambda b, be, bv: (b, 0)),
    )
    return pl.pallas_call(
        _ggemm_body,
        grid_spec=grid_spec,
        out_shape=jax.ShapeDtypeStruct((PADT, H), jnp.float32),
        compiler_params=pltpu.CompilerParams(
            dimension_semantics=("arbitrary",),
        ),
    )(blk_e, blk_v, disp, w1, w2, w3)


def _combine_sc(y, posa, posb):
    mesh = plsc.VectorSubcoreMesh(core_axis_name="c", subcore_axis_name="s")
    tpw = T // NW

    @functools.partial(
        pl.kernel,
        mesh=mesh,
        out_type=(
            jax.ShapeDtypeStruct((T, H), jnp.float32),
            jax.ShapeDtypeStruct((T, H), jnp.float32),
        ),
        scratch_types=[
            pltpu.VMEM((tpw,), jnp.int32),
            pltpu.VMEM((tpw, H), jnp.float32),
            pltpu.SemaphoreType.DMA,
        ],
    )
    def k(y_hbm, pa_hbm, pb_hbm, ya_hbm, yb_hbm, idx_v, buf_v, sem):
        w = lax.axis_index("s") * 2 + lax.axis_index("c")
        pltpu.sync_copy(pa_hbm.at[w], idx_v)
        pltpu.async_copy(y_hbm.at[idx_v], buf_v, sem).wait()
        pltpu.sync_copy(buf_v, ya_hbm.at[pl.ds(w * tpw, tpw)])
        pltpu.sync_copy(pb_hbm.at[w], idx_v)
        pltpu.async_copy(y_hbm.at[idx_v], buf_v, sem).wait()
        pltpu.sync_copy(buf_v, yb_hbm.at[pl.ds(w * tpw, tpw)])

    return k(y, posa, posb)


BTS = 512


def _final_body(x_ref, sg_ref, su_ref, sd_ref, ya_ref, yb_ref, wa_ref,
                wb_ref, o_ref):
    x = x_ref[...]
    g = jnp.dot(x, sg_ref[...], preferred_element_type=jnp.float32)
    u = jnp.dot(x, su_ref[...], preferred_element_type=jnp.float32)
    sh = jnp.dot(_silu(g) * u, sd_ref[...], preferred_element_type=jnp.float32)
    o_ref[...] = sh + wa_ref[...] * ya_ref[...] + wb_ref[...] * yb_ref[...]


def _final(x, sg, su, sd, ya, yb, wa, wb):
    nb = T // BTS
    return pl.pallas_call(
        _final_body,
        grid=(nb,),
        in_specs=[
            pl.BlockSpec((BTS, H), lambda t: (t, 0)),
            pl.BlockSpec((H, FS), lambda t: (0, 0)),
            pl.BlockSpec((H, FS), lambda t: (0, 0)),
            pl.BlockSpec((FS, H), lambda t: (0, 0)),
            pl.BlockSpec((BTS, H), lambda t: (t, 0)),
            pl.BlockSpec((BTS, H), lambda t: (t, 0)),
            pl.BlockSpec((BTS, 1), lambda t: (t, 0)),
            pl.BlockSpec((BTS, 1), lambda t: (t, 0)),
        ],
        out_specs=pl.BlockSpec((BTS, H), lambda t: (t, 0)),
        out_shape=jax.ShapeDtypeStruct((T, H), jnp.float32),
        compiler_params=pltpu.CompilerParams(
            dimension_semantics=("arbitrary",),
        ),
    )(x, sg, su, sd, ya, yb, wa, wb)


@jax.jit
def kernel(hidden_states, gate_w, gate_bias, w1, w2, w3, sg, su, sd):
    orig_shape = hidden_states.shape
    x = hidden_states.reshape(T, H)
    wa, wb, pos2, blk_e, blk_v = _routing(x, gate_w, gate_bias)
    pos3 = pos2.reshape(NW, 2, CW // 2)
    posa = pos2[0].reshape(NW, T // NW)
    posb = pos2[1].reshape(NW, T // NW)
    disp = _dispatch_sc(x, pos3)
    y = _ggemm(disp, blk_e.reshape(NBLK), blk_v.reshape(NBLK), w1, w2, w3)
    ya, yb = _combine_sc(y, posa, posb)
    out = _final(x, sg, su, sd, ya, yb, wa, wb)
    return out.reshape(orig_shape)


# speedup vs baseline: 1.2390x; 1.2390x over previous
import jax
import jax.numpy as jnp
from jax import lax
from jax.experimental import pallas as pl
from jax.experimental.pallas import tpu as pltpu

H = 1024
F = 512
E = 8
FS = 1024
FSB = FS // E
T = 2048


def _silu(v):
    return v * jax.nn.sigmoid(v)


def _moe_body(x_ref, gwt_ref, bias_ref, w1_ref, w2_ref, w3_ref,
              sg_ref, su_ref, sd_ref, o_ref, cmb_ref):
    e = pl.program_id(0)
    x = x_ref[...]

    @pl.when(e == 0)
    def _():
        logits = jnp.dot(x, gwt_ref[...], preferred_element_type=jnp.float32)
        scores = jax.nn.sigmoid(logits)
        sfc = scores + bias_ref[...]
        colE = lax.broadcasted_iota(jnp.int32, (T, E), 1)
        m1 = jnp.max(sfc, axis=1, keepdims=True)
        i1 = jnp.min(jnp.where(sfc == m1, colE, E), axis=1, keepdims=True)
        oh1 = (colE == i1).astype(jnp.float32)
        sfc2 = jnp.where(oh1 > 0, -jnp.inf, sfc)
        m2 = jnp.max(sfc2, axis=1, keepdims=True)
        i2 = jnp.min(jnp.where(sfc2 == m2, colE, E), axis=1, keepdims=True)
        oh2 = (colE == i2).astype(jnp.float32)
        s1 = jnp.sum(oh1 * scores, axis=1, keepdims=True)
        s2 = jnp.sum(oh2 * scores, axis=1, keepdims=True)
        den = s1 + s2 + 1e-20
        cmb_ref[...] = oh1 * (s1 / den) + oh2 * (s2 / den)

    h1 = jnp.dot(x, w1_ref[0], preferred_element_type=jnp.float32)
    h3 = jnp.dot(x, w3_ref[0], preferred_element_type=jnp.float32)
    y = jnp.dot(_silu(h1) * h3, w2_ref[0], preferred_element_type=jnp.float32)
    col = lax.broadcasted_iota(jnp.int32, (T, E), 1)
    ce = jnp.sum(jnp.where(col == e, cmb_ref[...], 0.0), axis=1,
                 keepdims=True)

    g = jnp.dot(x, sg_ref[...], preferred_element_type=jnp.float32)
    u = jnp.dot(x, su_ref[...], preferred_element_type=jnp.float32)
    sh = jnp.dot(_silu(g) * u, sd_ref[...], preferred_element_type=jnp.float32)

    @pl.when(e == 0)
    def _():
        o_ref[...] = ce * y + sh

    @pl.when(e != 0)
    def _():
        o_ref[...] += ce * y + sh


@jax.jit
def kernel(hidden_states, gate_w, gate_bias, w1, w2, w3, sg, su, sd):
    orig_shape = hidden_states.shape
    x = hidden_states.reshape(T, H)
    out = pl.pallas_call(
        _moe_body,
        grid=(E,),
        in_specs=[
            pl.BlockSpec((T, H), lambda e: (0, 0)),
            pl.BlockSpec((H, E), lambda e: (0, 0)),
            pl.BlockSpec((1, E), lambda e: (0, 0)),
            pl.BlockSpec((1, H, F), lambda e: (e, 0, 0)),
            pl.BlockSpec((1, F, H), lambda e: (e, 0, 0)),
            pl.BlockSpec((1, H, F), lambda e: (e, 0, 0)),
            pl.BlockSpec((H, FSB), lambda e: (0, e)),
            pl.BlockSpec((H, FSB), lambda e: (0, e)),
            pl.BlockSpec((FSB, H), lambda e: (e, 0)),
        ],
        out_specs=pl.BlockSpec((T, H), lambda e: (0, 0)),
        out_shape=jax.ShapeDtypeStruct((T, H), jnp.float32),
        scratch_shapes=[pltpu.VMEM((T, E), jnp.float32)],
        compiler_params=pltpu.CompilerParams(
            dimension_semantics=("arbitrary",),
        ),
    )(x, gate_w.T, gate_bias.reshape(1, E), w1, w2, w3, sg, su, sd)
    return out.reshape(orig_shape)


# speedup vs baseline: 1.3569x; 1.0952x over previous
import jax
import jax.numpy as jnp
from jax import lax
from jax.experimental import pallas as pl
from jax.experimental.pallas import tpu as pltpu

H = 1024
F = 512
E = 8
FS = 1024
T = 2048


def _silu(v):
    return v * jax.nn.sigmoid(v)


def _gate_shared_body(x_ref, gwt_ref, bias_ref, sg_ref, su_ref, sd_ref,
                      sh_ref, cmb_ref):
    x = x_ref[...]
    logits = jnp.dot(x, gwt_ref[...], preferred_element_type=jnp.float32)
    scores = jax.nn.sigmoid(logits)
    sfc = scores + bias_ref[...]
    colE = lax.broadcasted_iota(jnp.int32, (T, E), 1)
    m1 = jnp.max(sfc, axis=1, keepdims=True)
    i1 = jnp.min(jnp.where(sfc == m1, colE, E), axis=1, keepdims=True)
    oh1 = (colE == i1).astype(jnp.float32)
    sfc2 = jnp.where(oh1 > 0, -jnp.inf, sfc)
    m2 = jnp.max(sfc2, axis=1, keepdims=True)
    i2 = jnp.min(jnp.where(sfc2 == m2, colE, E), axis=1, keepdims=True)
    oh2 = (colE == i2).astype(jnp.float32)
    s1 = jnp.sum(oh1 * scores, axis=1, keepdims=True)
    s2 = jnp.sum(oh2 * scores, axis=1, keepdims=True)
    den = s1 + s2 + 1e-20
    cmb_ref[...] = oh1 * (s1 / den) + oh2 * (s2 / den)

    g = jnp.dot(x, sg_ref[...], preferred_element_type=jnp.float32)
    u = jnp.dot(x, su_ref[...], preferred_element_type=jnp.float32)
    sh_ref[...] = jnp.dot(_silu(g) * u, sd_ref[...],
                          preferred_element_type=jnp.float32)


def _gate_shared(x, gate_w, gate_bias, sg, su, sd):
    return pl.pallas_call(
        _gate_shared_body,
        out_shape=(
            jax.ShapeDtypeStruct((T, H), jnp.float32),
            jax.ShapeDtypeStruct((T, E), jnp.float32),
        ),
    )(x, gate_w.T, gate_bias.reshape(1, E), sg, su, sd)


def _moe_body(x_ref, cmb_ref, sh_ref, w1_ref, w2_ref, w3_ref, o_ref):
    e = pl.program_id(0)
    x = x_ref[...]
    h1 = jnp.dot(x, w1_ref[0], preferred_element_type=jnp.float32)
    h3 = jnp.dot(x, w3_ref[0], preferred_element_type=jnp.float32)
    col = lax.broadcasted_iota(jnp.int32, (T, E), 1)
    ce = jnp.sum(jnp.where(col == e, cmb_ref[...], 0.0), axis=1,
                 keepdims=True)
    p = ce * (_silu(h1) * h3)
    y = jnp.dot(p, w2_ref[0], preferred_element_type=jnp.float32)

    @pl.when(e == 0)
    def _():
        o_ref[...] = y + sh_ref[...]

    @pl.when(e != 0)
    def _():
        o_ref[...] += y


def _moe(x, combine, shared_out, w1, w2, w3):
    return pl.pallas_call(
        _moe_body,
        grid=(E,),
        in_specs=[
            pl.BlockSpec((T, H), lambda e: (0, 0)),
            pl.BlockSpec((T, E), lambda e: (0, 0)),
            pl.BlockSpec((T, H), lambda e: (0, 0)),
            pl.BlockSpec((1, H, F), lambda e: (e, 0, 0)),
            pl.BlockSpec((1, F, H), lambda e: (e, 0, 0)),
            pl.BlockSpec((1, H, F), lambda e: (e, 0, 0)),
        ],
        out_specs=pl.BlockSpec((T, H), lambda e: (0, 0)),
        out_shape=jax.ShapeDtypeStruct((T, H), jnp.float32),
        compiler_params=pltpu.CompilerParams(
            dimension_semantics=("arbitrary",),
        ),
    )(x, combine, shared_out, w1, w2, w3)


@jax.jit
def kernel(hidden_states, gate_w, gate_bias, w1, w2, w3, sg, su, sd):
    orig_shape = hidden_states.shape
    x = hidden_states.reshape(T, H)
    sh, combine = _gate_shared(x, gate_w, gate_bias, sg, su, sd)
    out = _moe(x, combine, sh, w1, w2, w3)
    return out.reshape(orig_shape)


# speedup vs baseline: 1.4413x; 1.0622x over previous
import jax
import jax.numpy as jnp
from jax import lax
from jax.experimental import pallas as pl
from jax.experimental.pallas import tpu as pltpu

H = 1024
F = 512
E = 8
FS = 1024
T = 2048


def _silu(v):
    return v * jax.nn.sigmoid(v)


def _gate_shared_body(x_ref, gwt_ref, bias_ref, sg_ref, su_ref, sd_ref,
                      sh_ref, cmb_ref):
    x = x_ref[...]
    logits = jnp.dot(x, gwt_ref[...], preferred_element_type=jnp.float32)
    scores = jax.nn.sigmoid(logits)
    sfc = scores + bias_ref[...]
    colE = lax.broadcasted_iota(jnp.int32, (T, E), 1)
    m1 = jnp.max(sfc, axis=1, keepdims=True)
    i1 = jnp.min(jnp.where(sfc == m1, colE, E), axis=1, keepdims=True)
    oh1 = (colE == i1).astype(jnp.float32)
    sfc2 = jnp.where(oh1 > 0, -jnp.inf, sfc)
    m2 = jnp.max(sfc2, axis=1, keepdims=True)
    i2 = jnp.min(jnp.where(sfc2 == m2, colE, E), axis=1, keepdims=True)
    oh2 = (colE == i2).astype(jnp.float32)
    s1 = jnp.sum(oh1 * scores, axis=1, keepdims=True)
    s2 = jnp.sum(oh2 * scores, axis=1, keepdims=True)
    den = s1 + s2 + 1e-20
    cmb_ref[...] = oh1 * (s1 / den) + oh2 * (s2 / den)

    g = jnp.dot(x, sg_ref[...], preferred_element_type=jnp.float32)
    u = jnp.dot(x, su_ref[...], preferred_element_type=jnp.float32)
    sh_ref[...] = jnp.dot(_silu(g) * u, sd_ref[...],
                          preferred_element_type=jnp.float32)


def _gate_shared(x, gate_w, gate_bias, sg, su, sd):
    return pl.pallas_call(
        _gate_shared_body,
        out_shape=(
            jax.ShapeDtypeStruct((T, H), jnp.float32),
            jax.ShapeDtypeStruct((T, E), jnp.float32),
        ),
    )(x, gate_w.T, gate_bias.reshape(1, E), sg, su, sd)


NCB = 4
CB = H // NCB


def _moe_body(x_ref, cmb_ref, sh_ref, w1_ref, w3_ref, w2a_ref, o_ref, p_ref):
    i = pl.program_id(0)

    @pl.when(i < E)
    def _():
        x = x_ref[...]
        h1 = jnp.dot(x, w1_ref[0], preferred_element_type=jnp.float32)
        h3 = jnp.dot(x, w3_ref[0], preferred_element_type=jnp.float32)
        col = lax.broadcasted_iota(jnp.int32, (T, E), 1)
        ce = jnp.sum(jnp.where(col == i, cmb_ref[...], 0.0), axis=1,
                     keepdims=True)
        p = (ce * (_silu(h1) * h3)).astype(jnp.bfloat16)
        p_ref[:, pl.ds(pl.multiple_of(i * F, F), F)] = p

    @pl.when(i >= E)
    def _():
        w2b = w2a_ref[...].astype(jnp.bfloat16)
        o_ref[...] = sh_ref[...] + jnp.dot(
            p_ref[...], w2b, preferred_element_type=jnp.float32)


def _moe(x, combine, shared_out, w1, w2, w3):
    w2a = w2.reshape(E * F, H)

    def _pblk(i):
        return jnp.maximum(i - E, 0)

    return pl.pallas_call(
        _moe_body,
        grid=(E + NCB,),
        in_specs=[
            pl.BlockSpec((T, H), lambda i: (0, 0)),
            pl.BlockSpec((T, E), lambda i: (0, 0)),
            pl.BlockSpec((T, CB), lambda i: (0, _pblk(i))),
            pl.BlockSpec((1, H, F), lambda i: (jnp.minimum(i, E - 1), 0, 0)),
            pl.BlockSpec((1, H, F), lambda i: (jnp.minimum(i, E - 1), 0, 0)),
            pl.BlockSpec((E * F, CB), lambda i: (0, _pblk(i))),
        ],
        out_specs=pl.BlockSpec((T, CB), lambda i: (0, _pblk(i))),
        out_shape=jax.ShapeDtypeStruct((T, H), jnp.float32),
        scratch_shapes=[pltpu.VMEM((T, E * F), jnp.bfloat16)],
        compiler_params=pltpu.CompilerParams(
            dimension_semantics=("arbitrary",),
            vmem_limit_bytes=100 * 1024 * 1024,
        ),
    )(x, combine, shared_out, w1, w3, w2a)


@jax.jit
def kernel(hidden_states, gate_w, gate_bias, w1, w2, w3, sg, su, sd):
    orig_shape = hidden_states.shape
    x = hidden_states.reshape(T, H)
    sh, combine = _gate_shared(x, gate_w, gate_bias, sg, su, sd)
    out = _moe(x, combine, sh, w1, w2, w3)
    return out.reshape(orig_shape)
